# baseline (speedup 1.0000x reference)
import functools

import jax
import jax.numpy as jnp
from jax import lax
from jax.experimental import pallas as pl
from jax.experimental.pallas import tpu as pltpu
from jax.experimental.pallas import tpu_sc as plsc

_N = 10000
_E = 320000
_S = 128
_DE = 16
_NA = 10240
_E2 = 327680
_NW = 32
_EPW = _E // _NW
_RPT = _NA // 16

_mesh = plsc.VectorSubcoreMesh(core_axis_name="c", subcore_axis_name="s")


@functools.partial(
    pl.kernel,
    out_type=jax.ShapeDtypeStruct((_NW, _NA), jnp.float32),
    mesh=_mesh,
    compiler_params=pltpu.CompilerParams(needs_layout_passes=False),
    scratch_types=[
        pltpu.VMEM((_NA,), jnp.float32),
        pltpu.VMEM((_EPW,), jnp.int32),
    ],
)
def _sc_hist(ei_hbm, out_hbm, hist_v, didx_v):
    c = lax.axis_index("c")
    s = lax.axis_index("s")
    w = c * 16 + s

    zero16 = jnp.zeros((16,), jnp.float32)

    def _zero(i, _):
        hist_v[pl.ds(i * 16, 16)] = zero16
        return ()

    lax.fori_loop(0, _NA // 16, _zero, (), unroll=4)

    pltpu.sync_copy(ei_hbm.at[pl.ds(_E + w * _EPW, _EPW)], didx_v)

    def _acc(i, _):
        idx = didx_v[pl.ds(i * 16, 16)]
        cnt, last = plsc.scan_count(idx)
        plsc.addupdate_scatter(hist_v, [idx], cnt.astype(jnp.float32), mask=last)
        return ()

    lax.fori_loop(0, _EPW // 16, _acc, (), unroll=4)

    pltpu.sync_copy(hist_v, out_hbm.at[w])


def _tc_pre_body(x_ref, hist_ref, wne_ref, bne_ref, ea_ref, wee_ref, bee_ref,
                 w3_ref, bout_ref, y_ref, dis_ref, r_ref):
    i = pl.program_id(0)
    nb = y_ref.shape[0]
    deg = jnp.sum(hist_ref[:, pl.ds(i * nb, nb)], axis=0) + 1.0
    dis = lax.rsqrt(deg)[:, None]
    xe = jnp.maximum(x_ref[...] @ wne_ref[...] + bne_ref[...], 0.0)
    y_ref[...] = xe * dis
    dis_ref[...] = dis
    ee = jnp.maximum(ea_ref[...] @ wee_ref[...] + bee_ref[...], 0.0)
    r_ref[...] = jnp.sum(ee * w3_ref[...], axis=1) + bout_ref[0, 0]


def _tc_pre(x, hist, wne, bne, ea, wee, bee, w3, bout):
    g = 20
    nb = _NA // g
    eb = _E2 // g
    return pl.pallas_call(
        _tc_pre_body,
        grid=(g,),
        in_specs=[
            pl.BlockSpec((nb, _S), lambda i: (i, 0)),
            pl.BlockSpec((_NW, _NA), lambda i: (0, 0)),
            pl.BlockSpec((_S, _S), lambda i: (0, 0)),
            pl.BlockSpec((1, _S), lambda i: (0, 0)),
            pl.BlockSpec((eb, _DE), lambda i: (i, 0)),
            pl.BlockSpec((_DE, _S), lambda i: (0, 0)),
            pl.BlockSpec((1, _S), lambda i: (0, 0)),
            pl.BlockSpec((1, _S), lambda i: (0, 0)),
            pl.BlockSpec((1, 1), lambda i: (0, 0)),
        ],
        out_specs=[
            pl.BlockSpec((nb, _S), lambda i: (i, 0)),
            pl.BlockSpec((nb, 1), lambda i: (i, 0)),
            pl.BlockSpec((eb,), lambda i: (i,)),
        ],
        out_shape=[
            jax.ShapeDtypeStruct((_NA, _S), jnp.float32),
            jax.ShapeDtypeStruct((_NA, 1), jnp.float32),
            jax.ShapeDtypeStruct((_E2,), jnp.float32),
        ],
    )(x, hist, wne, bne, ea, wee, bee, w3, bout)


_NAS = 10112
_RPT2 = _NAS // 16
_B = 80
_NB = _EPW // _B


@functools.partial(
    pl.kernel,
    out_type=jax.ShapeDtypeStruct((2, _NAS, _S), jnp.float32),
    mesh=_mesh,
    compiler_params=pltpu.CompilerParams(needs_layout_passes=False),
    scratch_types=[
        pltpu.VMEM_SHARED((_NAS, _S), jnp.float32),
        pltpu.VMEM((_EPW,), jnp.int32),
        pltpu.VMEM((_EPW,), jnp.int32),
        pltpu.VMEM((_B, _S), jnp.float32),
        pltpu.VMEM((_B, _S), jnp.float32),
        pltpu.SemaphoreType.DMA,
        pltpu.SemaphoreType.DMA,
        pltpu.SemaphoreType.DMA,
        pltpu.SemaphoreType.DMA,
    ],
)
def _sc_segsum(ei_hbm, y_hbm, zeros_hbm, acc_hbm,
               acc_sh, sidx_v, didx_v, ra, rb, gsa, gsb, ssa, ssb):
    c = lax.axis_index("c")
    s = lax.axis_index("s")
    w = c * 16 + s
    base = w * _EPW

    pltpu.sync_copy(zeros_hbm, acc_sh.at[pl.ds(s * _RPT2, _RPT2)])
    pltpu.sync_copy(ei_hbm.at[pl.ds(base, _EPW)], sidx_v)
    pltpu.sync_copy(ei_hbm.at[pl.ds(_E + base, _EPW)], didx_v)
    plsc.subcore_barrier()

    idx0 = didx_v.at[pl.ds(0, _B)]
    pltpu.async_copy(y_hbm.at[sidx_v.at[pl.ds(0, _B)]], ra, gsa)

    def _edge_batch(i, _):
        even = lax.rem(i, 2) == 0
        ni = i + 1

        @pl.when(ni < _NB)
        def _():
            @pl.when(even)
            def _():
                @pl.when(i >= 1)
                def _():
                    pltpu.make_async_copy(rb, acc_sh.at[idx0], ssb).wait()
                pltpu.async_copy(y_hbm.at[sidx_v.at[pl.ds(ni * _B, _B)]],
                                 rb, gsb)

            @pl.when(jnp.logical_not(even))
            def _():
                pltpu.make_async_copy(ra, acc_sh.at[idx0], ssa).wait()
                pltpu.async_copy(y_hbm.at[sidx_v.at[pl.ds(ni * _B, _B)]],
                                 ra, gsa)

        @pl.when(even)
        def _():
            pltpu.make_async_copy(y_hbm.at[sidx_v.at[pl.ds(0, _B)]],
                                  ra, gsa).wait()
            pltpu.async_copy(ra, acc_sh.at[didx_v.at[pl.ds(i * _B, _B)]],
                             ssa, add=True)

        @pl.when(jnp.logical_not(even))
        def _():
            pltpu.make_async_copy(y_hbm.at[sidx_v.at[pl.ds(0, _B)]],
                                  rb, gsb).wait()
            pltpu.async_copy(rb, acc_sh.at[didx_v.at[pl.ds(i * _B, _B)]],
                             ssb, add=True)

        return ()

    lax.fori_loop(0, _NB, _edge_batch, ())

    pltpu.make_async_copy(ra, acc_sh.at[idx0], ssa).wait()
    pltpu.make_async_copy(rb, acc_sh.at[idx0], ssb).wait()

    plsc.subcore_barrier()

    pltpu.sync_copy(acc_sh.at[pl.ds(s * _RPT2, _RPT2)],
                    acc_hbm.at[c].at[pl.ds(s * _RPT2, _RPT2)])


def _tc_gates_body(acc_ref, y_ref, dis_ref, wz_ref, bz_ref, lz_ref,
                   lbz_ref, wh_ref, bh_ref, lh_ref, lbh_ref, wout_ref,
                   p_ref, q_ref):
    agg = (acc_ref[0] + acc_ref[1] + y_ref[...]) * dis_ref[...]
    lz0 = lz_ref[0:_S, :]
    az = wz_ref[...] @ lz0
    cz = bz_ref[...] @ lz0 + lbz_ref[...]
    zg = jax.nn.sigmoid(agg @ az + cz)
    lh0 = lh_ref[0:_S, :]
    ah = wh_ref[...] @ lh0
    ch = bh_ref[...] @ lh0 + lbh_ref[...]
    ht = jnp.tanh(agg @ ah + ch)
    h = (1.0 - zg) * ht
    p_ref[...] = h @ wout_ref[0:_S, :]
    q_ref[...] = h @ wout_ref[_S:2 * _S, :]


def _tc_gates(acc, y, dis, wz, bz, lz, lbz, wh, bh, lh, lbh, wout):
    g = 5
    nb = _NA // g
    full = lambda a, b: pl.BlockSpec((a, b), lambda i: (0, 0))
    return pl.pallas_call(
        _tc_gates_body,
        grid=(g,),
        in_specs=[
            pl.BlockSpec((2, nb, _S), lambda i: (0, i, 0)),
            pl.BlockSpec((nb, _S), lambda i: (i, 0)),
            pl.BlockSpec((nb, 1), lambda i: (i, 0)),
            full(_S, _S), full(1, _S), full(2 * _S, _S), full(1, _S),
            full(_S, _S), full(1, _S), full(2 * _S, _S), full(1, _S),
            full(3 * _S, 1),
        ],
        out_specs=[
            pl.BlockSpec((nb, 1), lambda i: (i, 0)),
            pl.BlockSpec((nb, 1), lambda i: (i, 0)),
        ],
        out_shape=[
            jax.ShapeDtypeStruct((_NA, 1), jnp.float32),
            jax.ShapeDtypeStruct((_NA, 1), jnp.float32),
        ],
    )(acc, y, dis, wz, bz, lz, lbz, wh, bh, lh, lbh, wout)


@functools.partial(
    pl.kernel,
    out_type=jax.ShapeDtypeStruct((_E,), jnp.float32),
    mesh=_mesh,
    compiler_params=pltpu.CompilerParams(needs_layout_passes=False),
    scratch_types=[
        pltpu.VMEM((_NA,), jnp.float32),
        pltpu.VMEM((_NA,), jnp.float32),
        pltpu.VMEM((_EPW,), jnp.int32),
        pltpu.VMEM((_EPW,), jnp.int32),
        pltpu.VMEM((_EPW,), jnp.float32),
        pltpu.VMEM((_EPW,), jnp.float32),
    ],
)
def _sc_edge_out(ei_hbm, p_hbm, q_hbm, r_hbm, out_hbm,
                 p_v, q_v, sidx_v, didx_v, r_v, o_v):
    c = lax.axis_index("c")
    s = lax.axis_index("s")
    w = c * 16 + s
    base = w * _EPW

    pltpu.sync_copy(p_hbm, p_v)
    pltpu.sync_copy(q_hbm, q_v)
    pltpu.sync_copy(ei_hbm.at[pl.ds(base, _EPW)], sidx_v)
    pltpu.sync_copy(ei_hbm.at[pl.ds(_E + base, _EPW)], didx_v)
    pltpu.sync_copy(r_hbm.at[pl.ds(base, _EPW)], r_v)

    def _chunk(i, _):
        o = pl.ds(i * 16, 16)
        pv = plsc.load_gather(p_v, [sidx_v[o]])
        qv = plsc.load_gather(q_v, [didx_v[o]])
        o_v[o] = pv + qv + r_v[o]
        return ()

    lax.fori_loop(0, _EPW // 16, _chunk, (), unroll=4)

    pltpu.sync_copy(o_v, out_hbm.at[pl.ds(base, _EPW)])


def kernel(x, edge_index, edge_attr, W_ne, b_ne, W_ee, b_ee, Wz, bz, Lz, lbz,
           Wr, br, Lr, lbr, Wh, bh, Lh, lbh, W_out, b_out):
    ei = edge_index.astype(jnp.int32).reshape(2 * _E)

    hist = _sc_hist(ei)

    y, dis, r = _tc_pre(x, hist, W_ne, b_ne.reshape(1, _S),
                        edge_attr, W_ee, b_ee.reshape(1, _S),
                        W_out[2 * _S:, :].reshape(1, _S), b_out.reshape(1, 1))

    zeros_tile = jnp.zeros((_RPT2, _S), jnp.float32)
    acc = _sc_segsum(ei, y, zeros_tile)

    p, q = _tc_gates(
        acc, y, dis,
        Wz, bz.reshape(1, _S), Lz, lbz.reshape(1, _S),
        Wh, bh.reshape(1, _S), Lh, lbh.reshape(1, _S),
        W_out,
    )

    out = _sc_edge_out(ei, p.reshape(_NA), q.reshape(_NA), r)
    return out.reshape(_E, 1)


# speedup vs baseline: 1.1995x; 1.1995x over previous
import functools

import jax
import jax.numpy as jnp
from jax import lax
from jax.experimental import pallas as pl
from jax.experimental.pallas import tpu as pltpu
from jax.experimental.pallas import tpu_sc as plsc

_N = 10000
_E = 320000
_S = 128
_DE = 16
_NA = 10240
_E2 = 327680
_NW = 32
_EPW = _E // _NW
_RPT = _NA // 16

_mesh = plsc.VectorSubcoreMesh(core_axis_name="c", subcore_axis_name="s")


@functools.partial(
    pl.kernel,
    out_type=jax.ShapeDtypeStruct((_NW, _NA), jnp.float32),
    mesh=_mesh,
    compiler_params=pltpu.CompilerParams(needs_layout_passes=False),
    scratch_types=[
        pltpu.VMEM((_NA,), jnp.float32),
        pltpu.VMEM((_EPW,), jnp.int32),
    ],
)
def _sc_hist(ei_hbm, out_hbm, hist_v, didx_v):
    c = lax.axis_index("c")
    s = lax.axis_index("s")
    w = c * 16 + s

    zero16 = jnp.zeros((16,), jnp.float32)

    def _zero(i, _):
        hist_v[pl.ds(i * 16, 16)] = zero16
        return ()

    lax.fori_loop(0, _NA // 16, _zero, (), unroll=4)

    pltpu.sync_copy(ei_hbm.at[pl.ds(_E + w * _EPW, _EPW)], didx_v)

    def _acc(i, _):
        idx = didx_v[pl.ds(i * 16, 16)]
        cnt, last = plsc.scan_count(idx)
        plsc.addupdate_scatter(hist_v, [idx], cnt.astype(jnp.float32), mask=last)
        return ()

    lax.fori_loop(0, _EPW // 16, _acc, (), unroll=4)

    pltpu.sync_copy(hist_v, out_hbm.at[w])


def _tc_node_body(x_ref, hist_ref, wne_ref, bne_ref, y_ref, dis_ref):
    i = pl.program_id(0)
    nb = y_ref.shape[0]
    deg = jnp.sum(hist_ref[:, pl.ds(i * nb, nb)], axis=0) + 1.0
    dis = lax.rsqrt(deg)[:, None]
    xe = jnp.maximum(x_ref[...] @ wne_ref[...] + bne_ref[...], 0.0)
    y_ref[...] = xe * dis
    dis_ref[...] = dis


def _tc_node(x, hist, wne, bne):
    g = 10
    nb = _NA // g
    return pl.pallas_call(
        _tc_node_body,
        grid=(g,),
        in_specs=[
            pl.BlockSpec((nb, _S), lambda i: (i, 0)),
            pl.BlockSpec((_NW, _NA), lambda i: (0, 0)),
            pl.BlockSpec((_S, _S), lambda i: (0, 0)),
            pl.BlockSpec((1, _S), lambda i: (0, 0)),
        ],
        out_specs=[
            pl.BlockSpec((nb, _S), lambda i: (i, 0)),
            pl.BlockSpec((nb, 1), lambda i: (i, 0)),
        ],
        out_shape=[
            jax.ShapeDtypeStruct((_NA, _S), jnp.float32),
            jax.ShapeDtypeStruct((_NA, 1), jnp.float32),
        ],
    )(x, hist, wne, bne)


def _tc_edge_body(ea_ref, wee_ref, bee_ref, w3_ref, bout_ref, r_ref):
    ee = jnp.maximum(ea_ref[...] @ wee_ref[...] + bee_ref[...], 0.0)
    r_ref[...] = jnp.sum(ee * w3_ref[...], axis=1) + bout_ref[0, 0]


def _tc_edge(ea, wee, bee, w3, bout):
    g = 20
    eb = _E2 // g
    return pl.pallas_call(
        _tc_edge_body,
        grid=(g,),
        in_specs=[
            pl.BlockSpec((eb, _DE), lambda i: (i, 0)),
            pl.BlockSpec((_DE, _S), lambda i: (0, 0)),
            pl.BlockSpec((1, _S), lambda i: (0, 0)),
            pl.BlockSpec((1, _S), lambda i: (0, 0)),
            pl.BlockSpec((1, 1), lambda i: (0, 0)),
        ],
        out_specs=pl.BlockSpec((eb,), lambda i: (i,)),
        out_shape=jax.ShapeDtypeStruct((_E2,), jnp.float32),
    )(ea, wee, bee, w3, bout)


_NAS = 10112
_RPT2 = _NAS // 16
_B = 80
_NB = _EPW // _B


@functools.partial(
    pl.kernel,
    out_type=jax.ShapeDtypeStruct((2, _NAS, _S), jnp.float32),
    mesh=_mesh,
    compiler_params=pltpu.CompilerParams(needs_layout_passes=False),
    scratch_types=[
        pltpu.VMEM_SHARED((_NAS, _S), jnp.float32),
        pltpu.VMEM((_EPW,), jnp.int32),
        pltpu.VMEM((_EPW,), jnp.int32),
        pltpu.VMEM((_B, _S), jnp.float32),
        pltpu.VMEM((_B, _S), jnp.float32),
        pltpu.SemaphoreType.DMA,
        pltpu.SemaphoreType.DMA,
        pltpu.SemaphoreType.DMA,
        pltpu.SemaphoreType.DMA,
    ],
)
def _sc_segsum(ei_hbm, y_hbm, zeros_hbm, acc_hbm,
               acc_sh, sidx_v, didx_v, ra, rb, gsa, gsb, ssa, ssb):
    c = lax.axis_index("c")
    s = lax.axis_index("s")
    w = c * 16 + s
    base = w * _EPW

    pltpu.sync_copy(zeros_hbm, acc_sh.at[pl.ds(s * _RPT2, _RPT2)])
    pltpu.sync_copy(ei_hbm.at[pl.ds(base, _EPW)], sidx_v)
    pltpu.sync_copy(ei_hbm.at[pl.ds(_E + base, _EPW)], didx_v)
    plsc.subcore_barrier()

    idx0 = didx_v.at[pl.ds(0, _B)]
    pltpu.async_copy(y_hbm.at[sidx_v.at[pl.ds(0, _B)]], ra, gsa)

    def _edge_batch(i, _):
        even = lax.rem(i, 2) == 0
        ni = i + 1

        @pl.when(ni < _NB)
        def _():
            @pl.when(even)
            def _():
                @pl.when(i >= 1)
                def _():
                    pltpu.make_async_copy(rb, acc_sh.at[idx0], ssb).wait()
                pltpu.async_copy(y_hbm.at[sidx_v.at[pl.ds(ni * _B, _B)]],
                                 rb, gsb)

            @pl.when(jnp.logical_not(even))
            def _():
                pltpu.make_async_copy(ra, acc_sh.at[idx0], ssa).wait()
                pltpu.async_copy(y_hbm.at[sidx_v.at[pl.ds(ni * _B, _B)]],
                                 ra, gsa)

        @pl.when(even)
        def _():
            pltpu.make_async_copy(y_hbm.at[sidx_v.at[pl.ds(0, _B)]],
                                  ra, gsa).wait()
            pltpu.async_copy(ra, acc_sh.at[didx_v.at[pl.ds(i * _B, _B)]],
                             ssa, add=True)

        @pl.when(jnp.logical_not(even))
        def _():
            pltpu.make_async_copy(y_hbm.at[sidx_v.at[pl.ds(0, _B)]],
                                  rb, gsb).wait()
            pltpu.async_copy(rb, acc_sh.at[didx_v.at[pl.ds(i * _B, _B)]],
                             ssb, add=True)

        return ()

    lax.fori_loop(0, _NB, _edge_batch, ())

    pltpu.make_async_copy(ra, acc_sh.at[idx0], ssa).wait()
    pltpu.make_async_copy(rb, acc_sh.at[idx0], ssb).wait()

    plsc.subcore_barrier()

    pltpu.sync_copy(acc_sh.at[pl.ds(s * _RPT2, _RPT2)],
                    acc_hbm.at[c].at[pl.ds(s * _RPT2, _RPT2)])


def _tc_gates_body(acc_ref, y_ref, dis_ref, wz_ref, bz_ref, lz_ref,
                   lbz_ref, wh_ref, bh_ref, lh_ref, lbh_ref, wout_ref,
                   p_ref, q_ref):
    agg = (acc_ref[0] + acc_ref[1] + y_ref[...]) * dis_ref[...]
    lz0 = lz_ref[0:_S, :]
    az = wz_ref[...] @ lz0
    cz = bz_ref[...] @ lz0 + lbz_ref[...]
    zg = jax.nn.sigmoid(agg @ az + cz)
    lh0 = lh_ref[0:_S, :]
    ah = wh_ref[...] @ lh0
    ch = bh_ref[...] @ lh0 + lbh_ref[...]
    ht = jnp.tanh(agg @ ah + ch)
    h = (1.0 - zg) * ht
    p_ref[...] = h @ wout_ref[0:_S, :]
    q_ref[...] = h @ wout_ref[_S:2 * _S, :]


def _tc_gates(acc, y, dis, wz, bz, lz, lbz, wh, bh, lh, lbh, wout):
    g = 5
    nb = _NA // g
    full = lambda a, b: pl.BlockSpec((a, b), lambda i: (0, 0))
    return pl.pallas_call(
        _tc_gates_body,
        grid=(g,),
        in_specs=[
            pl.BlockSpec((2, nb, _S), lambda i: (0, i, 0)),
            pl.BlockSpec((nb, _S), lambda i: (i, 0)),
            pl.BlockSpec((nb, 1), lambda i: (i, 0)),
            full(_S, _S), full(1, _S), full(2 * _S, _S), full(1, _S),
            full(_S, _S), full(1, _S), full(2 * _S, _S), full(1, _S),
            full(3 * _S, 1),
        ],
        out_specs=[
            pl.BlockSpec((nb, 1), lambda i: (i, 0)),
            pl.BlockSpec((nb, 1), lambda i: (i, 0)),
        ],
        out_shape=[
            jax.ShapeDtypeStruct((_NA, 1), jnp.float32),
            jax.ShapeDtypeStruct((_NA, 1), jnp.float32),
        ],
    )(acc, y, dis, wz, bz, lz, lbz, wh, bh, lh, lbh, wout)


@functools.partial(
    pl.kernel,
    out_type=jax.ShapeDtypeStruct((_E,), jnp.float32),
    mesh=_mesh,
    compiler_params=pltpu.CompilerParams(needs_layout_passes=False),
    scratch_types=[
        pltpu.VMEM((_NA,), jnp.float32),
        pltpu.VMEM((_NA,), jnp.float32),
        pltpu.VMEM((_EPW,), jnp.int32),
        pltpu.VMEM((_EPW,), jnp.int32),
        pltpu.VMEM((_EPW,), jnp.float32),
        pltpu.VMEM((_EPW,), jnp.float32),
    ],
)
def _sc_edge_out(ei_hbm, p_hbm, q_hbm, r_hbm, out_hbm,
                 p_v, q_v, sidx_v, didx_v, r_v, o_v):
    c = lax.axis_index("c")
    s = lax.axis_index("s")
    w = c * 16 + s
    base = w * _EPW

    pltpu.sync_copy(p_hbm, p_v)
    pltpu.sync_copy(q_hbm, q_v)
    pltpu.sync_copy(ei_hbm.at[pl.ds(base, _EPW)], sidx_v)
    pltpu.sync_copy(ei_hbm.at[pl.ds(_E + base, _EPW)], didx_v)
    pltpu.sync_copy(r_hbm.at[pl.ds(base, _EPW)], r_v)

    def _chunk(i, _):
        o = pl.ds(i * 16, 16)
        pv = plsc.load_gather(p_v, [sidx_v[o]])
        qv = plsc.load_gather(q_v, [didx_v[o]])
        o_v[o] = pv + qv + r_v[o]
        return ()

    lax.fori_loop(0, _EPW // 16, _chunk, (), unroll=4)

    pltpu.sync_copy(o_v, out_hbm.at[pl.ds(base, _EPW)])


def kernel(x, edge_index, edge_attr, W_ne, b_ne, W_ee, b_ee, Wz, bz, Lz, lbz,
           Wr, br, Lr, lbr, Wh, bh, Lh, lbh, W_out, b_out):
    ei = edge_index.astype(jnp.int32).reshape(2 * _E)

    hist = _sc_hist(ei)

    y, dis = _tc_node(x, hist, W_ne, b_ne.reshape(1, _S))
    r = _tc_edge(edge_attr, W_ee, b_ee.reshape(1, _S),
                 W_out[2 * _S:, :].reshape(1, _S), b_out.reshape(1, 1))

    zeros_tile = jnp.zeros((_RPT2, _S), jnp.float32)
    acc = _sc_segsum(ei, y, zeros_tile)

    p, q = _tc_gates(
        acc, y, dis,
        Wz, bz.reshape(1, _S), Lz, lbz.reshape(1, _S),
        Wh, bh.reshape(1, _S), Lh, lbh.reshape(1, _S),
        W_out,
    )

    out = _sc_edge_out(ei, p.reshape(_NA), q.reshape(_NA), r)
    return out.reshape(_E, 1)
